# Initial kernel scaffold; baseline (speedup 1.0000x reference)
import jax
import jax.numpy as jnp
from jax.experimental import pallas as pl


def kernel(x, t, Wg1, bg1, Wg2, Wh1, Wh2, Wl1, Wl2, snr_threshold):
    raise NotImplementedError("write your pallas kernel here")


# speedup vs baseline: 3.3573x; 3.3573x over previous
import functools

import jax
import jax.numpy as jnp
from jax import lax
from jax.experimental import pallas as pl
from jax.experimental.pallas import tpu as pltpu

H = 768
E = 8
K = 2
INTER = 2048
IT = 512
ONE_BITS = 0x3F800000


def _cumsum_lanes(a):
    r, c = a.shape
    s = 1
    while s < c:
        shifted = jnp.concatenate(
            [jnp.zeros((r, s), a.dtype), a[:, : c - s]], axis=1)
        a = a + shifted
        s *= 2
    return a


def _routing_kernel(cap, xT_ref, wg1x_ref, wg1s_ref, bg1_ref, wg2_ref,
                    pos_ref, wts_ref, slot_ref, keepw_ref):
    T = xT_ref.shape[1]
    x = xT_ref[...]
    n = jnp.float32(H)
    s1 = jnp.sum(x, axis=0, keepdims=True)
    mean = s1 / n
    xc = x - mean
    var = jnp.sum(xc * xc, axis=0, keepdims=True) / (n - 1.0)
    std = jnp.sqrt(var)
    mn = jnp.min(x, axis=0, keepdims=True)
    mx = jnp.max(x, axis=0, keepdims=True)
    l2 = jnp.sqrt(jnp.sum(x * x, axis=0, keepdims=True))
    sp = jnp.sum((jnp.abs(x) < 1e-6).astype(jnp.float32), axis=0,
                 keepdims=True) / n
    statsT = jnp.concatenate(
        [mean, std, mn, mx, l2, sp, jnp.zeros((2, T), jnp.float32)], axis=0)

    h = (lax.dot_general(wg1x_ref[...], x, (((1,), (0,)), ((), ())),
                         preferred_element_type=jnp.float32)
         + lax.dot_general(wg1s_ref[...], statsT, (((1,), (0,)), ((), ())),
                           preferred_element_type=jnp.float32)
         + bg1_ref[...])
    h = 0.5 * h * (1.0 + lax.erf(h * 0.7071067811865476))
    logits = lax.dot_general(wg2_ref[...], h, (((1,), (0,)), ((), ())),
                             preferred_element_type=jnp.float32)

    m = jnp.max(logits, axis=0, keepdims=True)
    ex = jnp.exp(logits - m)
    p = ex / jnp.sum(ex, axis=0, keepdims=True)

    iota8 = lax.broadcasted_iota(jnp.int32, (E, T), 0)
    a1 = jnp.max(p, axis=0, keepdims=True)
    e1 = jnp.min(jnp.where(p == a1, iota8, E + 1), axis=0, keepdims=True)
    pmask = jnp.where(iota8 == e1, -jnp.inf, p)
    a2 = jnp.max(pmask, axis=0, keepdims=True)
    e2 = jnp.min(jnp.where(pmask == a2, iota8, E + 1), axis=0, keepdims=True)
    wsum = a1 + a2
    w1 = a1 / wsum
    w2 = a2 / wsum

    wfull = jnp.where(iota8 == e1, w1, jnp.where(iota8 == e2, w2, 0.0))
    valid = ((iota8 == e1) | (iota8 == e2)) & (wfull > 0.0)
    wbits = jnp.where(valid, lax.bitcast_convert_type(wfull, jnp.int32),
                      jnp.int32(-1))

    lo = jnp.zeros((E, 1), jnp.int32)
    hi = jnp.full((E, 1), ONE_BITS, jnp.int32)
    for _ in range(31):
        mid = (lo + hi) // 2
        cnt = jnp.sum((wbits > mid).astype(jnp.int32), axis=1, keepdims=True)
        small = cnt < cap
        upd = lo < hi
        hi = jnp.where(upd & small, mid, hi)
        lo = jnp.where(upd & (~small), mid + 1, lo)
    mstar = lo

    gt = wbits > mstar
    eq = wbits == mstar
    n_gt = jnp.sum(gt.astype(jnp.int32), axis=1, keepdims=True)
    quota = cap - n_gt
    eq_i = eq.astype(jnp.int32)
    eq_excl = _cumsum_lanes(eq_i) - eq_i
    keep = gt | (eq & (eq_excl < quota))
    keep_i = keep.astype(jnp.int32)
    slot = _cumsum_lanes(keep_i) - keep_i
    wkeep = jnp.where(keep, wfull, 0.0)

    slot_ref[...] = slot
    keepw_ref[...] = wkeep

    dummy = jnp.int32(E * cap)
    pos_full = jnp.where(keep, iota8 * cap + slot, dummy)
    sel1 = iota8 == e1
    sel2 = iota8 == e2
    pos0 = jnp.sum(jnp.where(sel1, pos_full, 0), axis=0, keepdims=True)
    pos1 = jnp.sum(jnp.where(sel2, pos_full, 0), axis=0, keepdims=True)
    w0 = jnp.sum(jnp.where(sel1 & keep, wfull, 0.0), axis=0, keepdims=True)
    w1k = jnp.sum(jnp.where(sel2 & keep, wfull, 0.0), axis=0, keepdims=True)
    zi = jnp.zeros((E - 2, T), jnp.int32)
    zf = jnp.zeros((E - 2, T), jnp.float32)
    pos_ref[...] = jnp.concatenate([pos0, pos1, zi], axis=0)
    wts_ref[...] = jnp.concatenate([w0, w1k, zf], axis=0)


def _dispatch_kernel(cap, xT_ref, slot_ref, keepw_ref, xbuf_ref):
    e = pl.program_id(0)
    T = xT_ref.shape[1]
    iota8 = lax.broadcasted_iota(jnp.int32, (E, T), 0)
    srow = jnp.sum(jnp.where(iota8 == e, slot_ref[...], 0), axis=0,
                   keepdims=True)
    krow = jnp.sum(jnp.where(iota8 == e, (keepw_ref[...] > 0.0)
                             .astype(jnp.int32), 0), axis=0, keepdims=True)
    iota_c = lax.broadcasted_iota(jnp.int32, (cap, T), 0)
    onehot = ((iota_c == srow) & (krow > 0)).astype(jnp.float32)
    xbuf_ref[...] = lax.dot_general(
        onehot, xT_ref[...], (((1,), (1,)), ((), ())),
        preferred_element_type=jnp.float32)


def _ffn_kernel(xbuf_ref, w1a_ref, w1b_ref, w2_ref, ybuf_ref):
    i = pl.program_id(1)
    xe = xbuf_ref[...]
    hg = lax.dot_general(xe, w1a_ref[0], (((1,), (1,)), ((), ())),
                         preferred_element_type=jnp.float32)
    hu = lax.dot_general(xe, w1b_ref[0], (((1,), (1,)), ((), ())),
                         preferred_element_type=jnp.float32)
    g = hg * jax.nn.sigmoid(hg) * hu
    contrib = lax.dot_general(g, w2_ref[0], (((1,), (1,)), ((), ())),
                              preferred_element_type=jnp.float32)

    @pl.when(i == 0)
    def _():
        ybuf_ref[...] = contrib

    @pl.when(i > 0)
    def _():
        ybuf_ref[...] = ybuf_ref[...] + contrib


def _combine_kernel(cap, ybuf_ref, slot_ref, keepw_ref, out_ref):
    e = pl.program_id(0)
    T = slot_ref.shape[1]
    iota8 = lax.broadcasted_iota(jnp.int32, (E, T), 0)
    srow = jnp.sum(jnp.where(iota8 == e, slot_ref[...], 0), axis=0,
                   keepdims=True)
    wrow = jnp.sum(jnp.where(iota8 == e, keepw_ref[...], 0.0), axis=0,
                   keepdims=True)
    iota_c = lax.broadcasted_iota(jnp.int32, (cap, T), 0)
    m = jnp.where(iota_c == srow, wrow, 0.0)
    contrib = lax.dot_general(m, ybuf_ref[...], (((0,), (0,)), ((), ())),
                              preferred_element_type=jnp.float32)

    @pl.when(e == 0)
    def _():
        out_ref[...] = contrib

    @pl.when(e > 0)
    def _():
        out_ref[...] = out_ref[...] + contrib


def _run_ffn(xbuf, w1, w2, cap):
    n_it = INTER // IT
    return pl.pallas_call(
        _ffn_kernel,
        grid=(E, n_it),
        in_specs=[
            pl.BlockSpec((cap, H), lambda e, i: (e, 0)),
            pl.BlockSpec((1, IT, H), lambda e, i: (e, i, 0)),
            pl.BlockSpec((1, IT, H), lambda e, i: (e, i + INTER // IT, 0)),
            pl.BlockSpec((1, H, IT), lambda e, i: (e, 0, i)),
        ],
        out_specs=pl.BlockSpec((cap, H), lambda e, i: (e, 0)),
        out_shape=jax.ShapeDtypeStruct((E * cap, H), jnp.float32),
    )(xbuf, w1, w1, w2)


def kernel(x, t, Wg1, bg1, Wg2, Wh1, Wh2, Wl1, Wl2, snr_threshold=0.5):
    B, N, C = x.shape
    T = B * N
    cap = int(1.25 * T / E)

    tokens = x.reshape(T, C)
    xT = tokens.T
    wg1x = Wg1[:, :H]
    wg1s = jnp.pad(Wg1[:, H:], ((0, 0), (0, 2)))
    bg1c = bg1.reshape(H // 2, 1)

    pos, wts, slot, keepw = pl.pallas_call(
        functools.partial(_routing_kernel, cap),
        out_shape=(
            jax.ShapeDtypeStruct((E, T), jnp.int32),
            jax.ShapeDtypeStruct((E, T), jnp.float32),
            jax.ShapeDtypeStruct((E, T), jnp.int32),
            jax.ShapeDtypeStruct((E, T), jnp.float32),
        ),
    )(xT, wg1x, wg1s, bg1c, Wg2)

    xbuf = pl.pallas_call(
        functools.partial(_dispatch_kernel, cap),
        grid=(E,),
        in_specs=[
            pl.BlockSpec((H, T), lambda e: (0, 0)),
            pl.BlockSpec((E, T), lambda e: (0, 0)),
            pl.BlockSpec((E, T), lambda e: (0, 0)),
        ],
        out_specs=pl.BlockSpec((cap, H), lambda e: (e, 0)),
        out_shape=jax.ShapeDtypeStruct((E * cap, H), jnp.float32),
    )(xT, slot, keepw)

    use_low = t.mean() >= snr_threshold
    ybuf = lax.cond(
        use_low,
        lambda xb: _run_ffn(xb, Wl1, Wl2, cap),
        lambda xb: _run_ffn(xb, Wh1, Wh2, cap),
        xbuf)

    out = pl.pallas_call(
        functools.partial(_combine_kernel, cap),
        grid=(E,),
        in_specs=[
            pl.BlockSpec((cap, H), lambda e: (e, 0)),
            pl.BlockSpec((E, T), lambda e: (0, 0)),
            pl.BlockSpec((E, T), lambda e: (0, 0)),
        ],
        out_specs=pl.BlockSpec((T, H), lambda e: (0, 0)),
        out_shape=jax.ShapeDtypeStruct((T, H), jnp.float32),
    )(ybuf, slot, keepw)

    y = out.reshape(B, N, C)
    return (y, jnp.zeros((), dtype=jnp.float32))
